# baseline (speedup 1.0000x reference)
import functools

import jax
import jax.numpy as jnp
from jax import lax
from jax.experimental import pallas as pl
from jax.experimental.pallas import tpu as pltpu
from jax.experimental.pallas import tpu_sc as plsc

N = 16384
NEWS_DIM = 64
CAT_DIM = 16
D = NEWS_DIM + CAT_DIM
NUM_NEWS = 1000000
NUM_CATEGORIES = 1000
CH = 128
R_TC = 2048
T = R_TC * 2 * 153


def _tc_linearize_body(lo_ref, hi_ref, out_ref):
    out_ref[...] = jnp.concatenate([lo_ref[...], hi_ref[...]], axis=1)


@functools.lru_cache(maxsize=1)
def _build_tc_linearize():
    nb = T // 2 // R_TC
    return pl.pallas_call(
        _tc_linearize_body,
        grid=(nb,),
        in_specs=[pl.BlockSpec((R_TC, NEWS_DIM), lambda c: (c, 0)),
                  pl.BlockSpec((R_TC, NEWS_DIM), lambda c, _nb=nb: (c + _nb, 0))],
        out_specs=pl.BlockSpec((R_TC, 2 * NEWS_DIM), lambda c: (c, 0)),
        out_shape=jax.ShapeDtypeStruct((T // 2, 2 * NEWS_DIM), jnp.float32),
    )


@functools.lru_cache(maxsize=1)
def _build_sc_b():
    info = plsc.get_sparse_core_info()
    nc, ns = info.num_cores, info.num_subcores
    nw = nc * ns
    bpw = N // nw
    cat_words = NUM_CATEGORIES * CAT_DIM
    mesh = plsc.VectorSubcoreMesh(core_axis_name="c", subcore_axis_name="s")

    @functools.partial(
        pl.kernel,
        mesh=mesh,
        out_type=jax.ShapeDtypeStruct((N * D,), jnp.float32),
        scratch_types=[
            pltpu.VMEM((bpw,), jnp.int32),
            pltpu.VMEM((bpw,), jnp.int32),
            pltpu.VMEM((bpw,), jnp.float32),
            pltpu.VMEM((bpw, NEWS_DIM), jnp.float32),
            pltpu.VMEM((cat_words,), jnp.float32),
            pltpu.VMEM((bpw * D,), jnp.float32),
            pltpu.SemaphoreType.DMA,
            pltpu.SemaphoreType.DMA,
        ],
    )
    def sc_b(news_ids_hbm, cat_ids_hbm, mask_hbm, news_tab_hbm,
             cat_tab_hbm, out_hbm,
             nids_v, cids_v, mask_v, news_v, cat_tab_v, out_v, sem, csem):
        wid = lax.axis_index("s") * nc + lax.axis_index("c")
        base = wid * bpw
        pltpu.sync_copy(news_ids_hbm.at[pl.ds(base, bpw)], nids_v)
        pltpu.sync_copy(cat_ids_hbm.at[pl.ds(base, bpw)], cids_v)
        pltpu.sync_copy(mask_hbm.at[pl.ds(base, bpw)], mask_v)
        ccopy = pltpu.async_copy(cat_tab_hbm, cat_tab_v, csem)

        def fire(g, cnt):
            ids16 = nids_v[pl.ds(g * 16, 16)]
            for k in range(16):
                i = g * 16 + k
                r = ids16[k]
                p = r >= T

                @pl.when(p)
                def _():
                    pltpu.async_copy(news_tab_hbm.at[pl.ds(r, 1)],
                                     news_v.at[pl.ds(i, 1)], sem)
                cnt = cnt + jnp.where(p, jnp.int32(1), jnp.int32(0))
            return cnt
        cnt = lax.fori_loop(0, bpw // 16, fire, jnp.int32(0))

        def drain(i, carry):
            pltpu.make_async_copy(news_tab_hbm.at[pl.ds(0, 1)],
                                  news_v.at[pl.ds(0, 1)], sem).wait()
            return carry
        lax.fori_loop(0, cnt, drain, 0)
        ccopy.wait()

        def body(g, carry):
            cpos16 = cids_v[pl.ds(g * 16, 16)] * CAT_DIM
            m16 = mask_v[pl.ds(g * 16, 16)]
            for k in range(16):
                i = g * 16 + k
                cp = cpos16[k]
                m = m16[k]
                obase = i * D
                for j in range(NEWS_DIM // 16):
                    out_v[pl.ds(obase + j * 16, 16)] = (
                        news_v[i, pl.ds(j * 16, 16)] * m)
                out_v[pl.ds(obase + NEWS_DIM, CAT_DIM)] = (
                    cat_tab_v[pl.ds(cp, CAT_DIM)] * m)
            return carry
        lax.fori_loop(0, bpw // 16, body, 0)

        pltpu.sync_copy(out_v, out_hbm.at[pl.ds(base * D, bpw * D)])

    return sc_b


@functools.lru_cache(maxsize=1)
def _build_sc_a():
    info = plsc.get_sparse_core_info()
    nc, ns = info.num_cores, info.num_subcores
    nw = nc * ns
    bpw = N // nw
    n_chunks = bpw // CH
    cat_words = NUM_CATEGORIES * CAT_DIM
    mesh = plsc.VectorSubcoreMesh(core_axis_name="c", subcore_axis_name="s")

    @functools.partial(
        pl.kernel,
        mesh=mesh,
        out_type=jax.ShapeDtypeStruct((N * D,), jnp.float32),
        scratch_types=[
            pltpu.VMEM((bpw,), jnp.int32),
            pltpu.VMEM((bpw,), jnp.int32),
            pltpu.VMEM((bpw,), jnp.float32),
            pltpu.VMEM((n_chunks, CH), jnp.int32),
            pltpu.VMEM((bpw, 2 * NEWS_DIM), jnp.float32),
            pltpu.VMEM((cat_words,), jnp.float32),
            pltpu.VMEM((bpw * D,), jnp.float32),
            pltpu.SemaphoreType.DMA,
        ],
    )
    def sc_a(news_ids_hbm, cat_ids_hbm, mask_hbm, lin_tab_hbm,
             cat_tab_hbm, out_hbm,
             nids_v, cids_v, mask_v, nidx_v, news_v, cat_tab_v, out_v, sem):
        wid = lax.axis_index("s") * nc + lax.axis_index("c")
        base = wid * bpw
        pltpu.sync_copy(news_ids_hbm.at[pl.ds(base, bpw)], nids_v)
        pltpu.sync_copy(cat_ids_hbm.at[pl.ds(base, bpw)], cids_v)
        pltpu.sync_copy(mask_hbm.at[pl.ds(base, bpw)], mask_v)

        t2 = T // 2
        for c in range(n_chunks):
            for v in range(CH // 16):
                idc = jnp.minimum(nids_v[pl.ds(c * CH + v * 16, 16)], T - 1)
                nidx_v[c, pl.ds(v * 16, 16)] = jnp.where(
                    idc < t2, idc, idc - t2)

        copies = [pltpu.async_copy(cat_tab_hbm, cat_tab_v, sem)]
        for c in range(n_chunks):
            copies.append(pltpu.async_copy(
                lin_tab_hbm.at[nidx_v.at[c]],
                news_v.at[pl.ds(c * CH, CH)], sem))
        for c in copies:
            c.wait()

        def body(g, carry):
            idc16 = jnp.minimum(nids_v[pl.ds(g * 16, 16)], T - 1)
            off16 = jnp.where(idc16 < T // 2, 0, NEWS_DIM)
            cpos16 = cids_v[pl.ds(g * 16, 16)] * CAT_DIM
            m16 = mask_v[pl.ds(g * 16, 16)]
            for k in range(16):
                i = g * 16 + k
                off = off16[k]
                cp = cpos16[k]
                m = m16[k]
                obase = i * D
                for j in range(NEWS_DIM // 16):
                    out_v[pl.ds(obase + j * 16, 16)] = (
                        news_v[i, pl.ds(off + j * 16, 16)] * m)
                out_v[pl.ds(obase + NEWS_DIM, CAT_DIM)] = (
                    cat_tab_v[pl.ds(cp, CAT_DIM)] * m)
            return carry
        lax.fori_loop(0, bpw // 16, body, 0)

        pltpu.sync_copy(out_v, out_hbm.at[pl.ds(base * D, bpw * D)])

    return sc_a


def kernel(news_ids, category_ids, delta_t, mask, news_table, category_table):
    cat_flat = jnp.reshape(category_table, (NUM_CATEGORIES * CAT_DIM,))
    out_b = _build_sc_b()(news_ids, category_ids, mask, news_table, cat_flat)
    lin = _build_tc_linearize()(news_table, news_table)
    out_a = _build_sc_a()(news_ids, category_ids, mask, lin, cat_flat)
    sel = (news_ids < T)[:, None]
    X = jnp.where(sel, jnp.reshape(out_a, (N, D)), jnp.reshape(out_b, (N, D)))
    return (X, delta_t)


# speedup vs baseline: 2.1675x; 2.1675x over previous
import functools

import jax
import jax.numpy as jnp
from jax import lax
from jax.experimental import pallas as pl
from jax.experimental.pallas import tpu as pltpu
from jax.experimental.pallas import tpu_sc as plsc

N = 16384
NEWS_DIM = 64
CAT_DIM = 16
D = NEWS_DIM + CAT_DIM
NUM_NEWS = 1000000
NUM_CATEGORIES = 1000


@functools.lru_cache(maxsize=1)
def _build_sc_kernel():
    info = plsc.get_sparse_core_info()
    nc, ns = info.num_cores, info.num_subcores
    nw = nc * ns
    bpw = N // nw
    half = bpw // 2
    cat_words = NUM_CATEGORIES * CAT_DIM
    mesh = plsc.VectorSubcoreMesh(core_axis_name="c", subcore_axis_name="s")

    @functools.partial(
        pl.kernel,
        mesh=mesh,
        out_type=jax.ShapeDtypeStruct((N * D,), jnp.float32),
        scratch_types=[
            pltpu.VMEM((bpw,), jnp.int32),
            pltpu.VMEM((bpw,), jnp.int32),
            pltpu.VMEM((bpw,), jnp.float32),
            pltpu.VMEM((half, NEWS_DIM), jnp.float32),
            pltpu.VMEM((half, NEWS_DIM), jnp.float32),
            pltpu.VMEM((cat_words,), jnp.float32),
            pltpu.VMEM((bpw * D,), jnp.float32),
            pltpu.SemaphoreType.DMA,
            pltpu.SemaphoreType.DMA,
            pltpu.SemaphoreType.DMA,
        ],
    )
    def sc_kernel(news_ids_hbm, cat_ids_hbm, mask_hbm, news_tab_hbm,
                  cat_tab_hbm, out_hbm,
                  nids_v, cids_v, mask_v, news_v0, news_v1, cat_tab_v, out_v,
                  sem0, sem1, csem):
        wid = lax.axis_index("s") * nc + lax.axis_index("c")
        base = wid * bpw
        pltpu.sync_copy(news_ids_hbm.at[pl.ds(base, bpw)], nids_v)
        pltpu.sync_copy(cat_ids_hbm.at[pl.ds(base, bpw)], cids_v)
        pltpu.sync_copy(mask_hbm.at[pl.ds(base, bpw)], mask_v)
        ccopy = pltpu.async_copy(cat_tab_hbm, cat_tab_v, csem)

        bufs = (news_v0, news_v1)
        sems = (sem0, sem1)

        def fire(g, carry):
            ids16 = nids_v[pl.ds(g * 16, 16)]
            for k in range(16):
                r = ids16[k]
                pltpu.async_copy(news_tab_hbm.at[pl.ds(r, 1)],
                                 bufs[k % 2].at[pl.ds(g * 8 + k // 2, 1)],
                                 sems[k % 2])
            return carry
        lax.fori_loop(0, bpw // 16, fire, 0)
        pltpu.make_async_copy(news_tab_hbm.at[pl.ds(0, half)], news_v0,
                              sem0).wait()
        pltpu.make_async_copy(news_tab_hbm.at[pl.ds(0, half)], news_v1,
                              sem1).wait()
        ccopy.wait()

        def body(g, carry):
            cpos16 = cids_v[pl.ds(g * 16, 16)] * CAT_DIM
            m16 = mask_v[pl.ds(g * 16, 16)]
            for k in range(16):
                i = g * 16 + k
                cp = cpos16[k]
                m = m16[k]
                obase = i * D
                src = bufs[k % 2]
                row = g * 8 + k // 2
                for j in range(NEWS_DIM // 16):
                    out_v[pl.ds(obase + j * 16, 16)] = (
                        src[row, pl.ds(j * 16, 16)] * m)
                out_v[pl.ds(obase + NEWS_DIM, CAT_DIM)] = (
                    cat_tab_v[pl.ds(cp, CAT_DIM)] * m)
            return carry
        lax.fori_loop(0, bpw // 16, body, 0)

        pltpu.sync_copy(out_v, out_hbm.at[pl.ds(base * D, bpw * D)])

    return sc_kernel


def kernel(news_ids, category_ids, delta_t, mask, news_table, category_table):
    sc = _build_sc_kernel()
    cat_flat = jnp.reshape(category_table, (NUM_CATEGORIES * CAT_DIM,))
    out = sc(news_ids, category_ids, mask, news_table, cat_flat)
    return (jnp.reshape(out, (N, D)), delta_t)


# speedup vs baseline: 2.1677x; 1.0001x over previous
import functools

import jax
import jax.numpy as jnp
from jax import lax
from jax.experimental import pallas as pl
from jax.experimental.pallas import tpu as pltpu
from jax.experimental.pallas import tpu_sc as plsc

N = 16384
NEWS_DIM = 64
CAT_DIM = 16
D = NEWS_DIM + CAT_DIM
NUM_NEWS = 1000000
NUM_CATEGORIES = 1000


@functools.lru_cache(maxsize=1)
def _build_sc_kernel():
    info = plsc.get_sparse_core_info()
    nc, ns = info.num_cores, info.num_subcores
    nw = nc * ns
    bpw = N // nw
    half = bpw // 2
    cat_words = NUM_CATEGORIES * CAT_DIM
    mesh = plsc.VectorSubcoreMesh(core_axis_name="c", subcore_axis_name="s")

    @functools.partial(
        pl.kernel,
        mesh=mesh,
        out_type=jax.ShapeDtypeStruct((N * D,), jnp.float32),
        scratch_types=[
            pltpu.VMEM((bpw,), jnp.int32),
            pltpu.VMEM((bpw,), jnp.int32),
            pltpu.VMEM((bpw,), jnp.float32),
            pltpu.VMEM((half, NEWS_DIM), jnp.float32),
            pltpu.VMEM((half, NEWS_DIM), jnp.float32),
            pltpu.VMEM((cat_words,), jnp.float32),
            pltpu.VMEM((bpw * D,), jnp.float32),
            pltpu.SemaphoreType.DMA,
            pltpu.SemaphoreType.DMA,
            pltpu.SemaphoreType.DMA,
            pltpu.SemaphoreType.DMA,
            pltpu.SemaphoreType.DMA,
        ],
    )
    def sc_kernel(news_ids_hbm, cat_ids_hbm, mask_hbm, news_tab_hbm,
                  cat_tab_hbm, out_hbm,
                  nids_v, cids_v, mask_v, news_v0, news_v1, cat_tab_v, out_v,
                  sem0, sem1, sem2, sem3, csem):
        wid = lax.axis_index("s") * nc + lax.axis_index("c")
        base = wid * bpw
        pltpu.sync_copy(news_ids_hbm.at[pl.ds(base, bpw)], nids_v)
        pltpu.sync_copy(cat_ids_hbm.at[pl.ds(base, bpw)], cids_v)
        pltpu.sync_copy(mask_hbm.at[pl.ds(base, bpw)], mask_v)
        ccopy = pltpu.async_copy(cat_tab_hbm, cat_tab_v, csem)

        bufs = (news_v0, news_v1)
        sems = (sem0, sem1, sem2, sem3)
        ng = bpw // 16

        def fire(g, sem):
            ids16 = nids_v[pl.ds(g * 16, 16)]
            for k in range(16):
                r = ids16[k]
                pltpu.async_copy(news_tab_hbm.at[pl.ds(r, 1)],
                                 bufs[k % 2].at[pl.ds(g * 8 + k // 2, 1)],
                                 sem)

        def drain(sem):
            pltpu.make_async_copy(news_tab_hbm.at[pl.ds(0, 8)],
                                  news_v0.at[pl.ds(0, 8)], sem).wait()
            pltpu.make_async_copy(news_tab_hbm.at[pl.ds(0, 8)],
                                  news_v1.at[pl.ds(0, 8)], sem).wait()

        def compute(g):
            cpos16 = cids_v[pl.ds(g * 16, 16)] * CAT_DIM
            m16 = mask_v[pl.ds(g * 16, 16)]
            for k in range(16):
                i = g * 16 + k
                cp = cpos16[k]
                m = m16[k]
                obase = i * D
                src = bufs[k % 2]
                row = g * 8 + k // 2
                for j in range(NEWS_DIM // 16):
                    out_v[pl.ds(obase + j * 16, 16)] = (
                        src[row, pl.ds(j * 16, 16)] * m)
                out_v[pl.ds(obase + NEWS_DIM, CAT_DIM)] = (
                    cat_tab_v[pl.ds(cp, CAT_DIM)] * m)

        ccopy.wait()

        nsup = ng // 4

        def super_fire(w):
            for kk in range(4):
                fire(w * 4 + kk, sems[kk])

        def super_drain(_):
            for kk in range(4):
                drain(sems[kk])

        def super_compute(w):
            for kk in range(4):
                compute(w * 4 + kk)

        def step(w, carry):
            @pl.when(w >= 1)
            def _():
                super_drain(w - 1)
            super_fire(w)
            @pl.when(w >= 1)
            def _():
                super_compute(w - 1)
            return carry
        lax.fori_loop(0, nsup, step, 0)
        super_drain(nsup - 1)
        super_compute(nsup - 1)

        pltpu.sync_copy(out_v, out_hbm.at[pl.ds(base * D, bpw * D)])

    return sc_kernel


def kernel(news_ids, category_ids, delta_t, mask, news_table, category_table):
    sc = _build_sc_kernel()
    cat_flat = jnp.reshape(category_table, (NUM_CATEGORIES * CAT_DIM,))
    out = sc(news_ids, category_ids, mask, news_table, cat_flat)
    return (jnp.reshape(out, (N, D)), delta_t)
